# Initial kernel scaffold; baseline (speedup 1.0000x reference)
import jax
import jax.numpy as jnp
from jax.experimental import pallas as pl


def kernel(step, lprobs, scores):
    raise NotImplementedError("write your pallas kernel here")


# speedup vs baseline: 2.7817x; 2.7817x over previous
import functools

import jax
import jax.numpy as jnp
from jax import lax
from jax.experimental import pallas as pl

_NBLK = 50
_BLK = 2000


def _row_kernel(lp_ref, sc_ref, vals_ref, idx_ref, beams_ref):
    _NEG = jnp.float32(-jnp.inf)
    _BIG = jnp.int32(1 << 30)
    sc = sc_ref[0, 0, :]
    M = jnp.max(lp_ref[0] + sc[:, None, None], axis=2)
    jio = lax.broadcasted_iota(jnp.int32, (8, _NBLK), 0)
    bio = lax.broadcasted_iota(jnp.int32, (8, _NBLK), 1)
    ordv = (jio // 2) * _NBLK + bio
    lio = lax.broadcasted_iota(jnp.int32, (1, 1, _BLK), 2)
    iota8 = lax.broadcasted_iota(jnp.int32, (8,), 0)
    half = jnp.float32(0.5)

    def pick8(Mg, g, pens):
        picks, rem = [], []
        for _ in range(8):
            m = jnp.max(Mg)
            kb = jnp.min(jnp.where(Mg == m, ordv, _BIG))
            s_ = kb // _NBLK
            b_ = kb % _NBLK
            j_ = s_ * 2 + g
            scj = jnp.max(jnp.where(iota8 == j_, sc, _NEG))
            blk = lp_ref[0, pl.ds(j_, 1), pl.ds(b_, 1), :] + scj
            for (bu, lu) in pens:
                blk = blk - jnp.where((bu == b_) & (lio == lu), half, jnp.float32(0.0))
            for (jp, bp, lp_) in rem:
                blk = jnp.where((jp == j_) & (bp == b_) & (lio == lp_), _NEG, blk)
            mv = jnp.max(blk)
            l_ = jnp.min(jnp.where(blk == mv, lio, _BIG))
            newm = jnp.max(jnp.where(lio == l_, _NEG, blk))
            Mg = jnp.where((jio == j_) & (bio == b_), newm, Mg)
            rem.append((j_, b_, l_))
            picks.append((mv, b_ * _BLK + l_, j_, b_, l_))
        return picks

    p0 = pick8(jnp.where(jio % 2 == 0, M, _NEG), 0, [])
    pens = [(pk[3], pk[4]) for pk in p0]

    odd3 = lax.broadcasted_iota(jnp.int32, (8, 1, _BLK), 0) % 2 == 1
    lio8 = lax.broadcasted_iota(jnp.int32, (8, 1, _BLK), 2)
    Mfix = M
    for (bt, _lt) in pens:
        sl = lp_ref[0, :, pl.ds(bt, 1), :] + sc[:, None, None]
        for (bu, lu) in pens:
            sl = sl - jnp.where(odd3 & (bu == bt) & (lio8 == lu), half, jnp.float32(0.0))
        newm = jnp.max(sl, axis=2)
        Mfix = jnp.where((jio % 2 == 1) & (bio == bt), newm, Mfix)

    p1 = pick8(jnp.where(jio % 2 == 1, Mfix, _NEG), 1, pens)

    i16 = lax.broadcasted_iota(jnp.int32, (1, 1, 16), 2)
    vv = jnp.zeros((1, 1, 16), jnp.float32)
    iv = jnp.zeros((1, 1, 16), jnp.int32)
    bv = jnp.zeros((1, 1, 16), jnp.int32)
    for k in range(8):
        for g, pk in ((0, p0[k]), (1, p1[k])):
            slot = 2 * k + g
            vv = jnp.where(i16 == slot, pk[0], vv)
            iv = jnp.where(i16 == slot, pk[1], iv)
            bv = jnp.where(i16 == slot, pk[2], bv)
    vals_ref[...] = vv
    idx_ref[...] = iv
    beams_ref[...] = bv


@functools.partial(jax.jit, static_argnames=())
def _run(lp4, sc3):
    bsz = lp4.shape[0]
    grid = (bsz,)
    out = pl.pallas_call(
        _row_kernel,
        grid=grid,
        in_specs=[
            pl.BlockSpec((1, 8, _NBLK, _BLK), lambda i: (i, 0, 0, 0)),
            pl.BlockSpec((1, 1, 8), lambda i: (i, 0, 0)),
        ],
        out_specs=[
            pl.BlockSpec((1, 1, 16), lambda i: (i, 0, 0)),
            pl.BlockSpec((1, 1, 16), lambda i: (i, 0, 0)),
            pl.BlockSpec((1, 1, 16), lambda i: (i, 0, 0)),
        ],
        out_shape=[
            jax.ShapeDtypeStruct((bsz, 1, 16), jnp.float32),
            jax.ShapeDtypeStruct((bsz, 1, 16), jnp.int32),
            jax.ShapeDtypeStruct((bsz, 1, 16), jnp.int32),
        ],
    )(lp4, sc3)
    return out


def kernel(step, lprobs, scores):
    bsz, beam_size, vocab = lprobs.shape
    lp4 = lprobs.reshape(bsz, beam_size, _NBLK, _BLK)
    sc = lax.dynamic_slice_in_dim(scores, step - 1, 1, axis=2)
    sc3 = sc.reshape(bsz, 1, beam_size)
    vv, iv, bv = _run(lp4, sc3)
    return (vv.reshape(bsz, 16), iv.reshape(bsz, 16), bv.reshape(bsz, 16))


# speedup vs baseline: 4.3384x; 1.5596x over previous
import functools

import jax
import jax.numpy as jnp
from jax import lax
from jax.experimental import pallas as pl
from jax.experimental.pallas import tpu as pltpu

_NBLK = 50
_BLK = 2000
_R = 4


def _rows_kernel(lp_ref, sc_ref, vals_ref, idx_ref, beams_ref, *x_refs):
    _NEG = jnp.float32(-jnp.inf)
    _BIG = jnp.int32(1 << 30)
    half = jnp.float32(0.5)
    jio = lax.broadcasted_iota(jnp.int32, (8, _NBLK), 0)
    bio = lax.broadcasted_iota(jnp.int32, (8, _NBLK), 1)
    ordv = (jio // 2) * _NBLK + bio
    lio = lax.broadcasted_iota(jnp.int32, (1, 1, _BLK), 2)
    lio8 = lax.broadcasted_iota(jnp.int32, (8, 1, _BLK), 2)
    odd3 = lax.broadcasted_iota(jnp.int32, (8, 1, _BLK), 0) % 2 == 1

    Ms = []
    for r in range(_R):
        xv = lp_ref[r] + sc_ref[r, 0, :][:, None, None]
        x_refs[r][...] = xv
        Ms.append(jnp.max(xv, axis=2))

    def pick_round(Mgs, g, picks):
        for r in range(_R):
            m = jnp.max(Mgs[r])
            kb = jnp.min(jnp.where(Mgs[r] == m, ordv, _BIG))
            s_ = kb // _NBLK
            b_ = kb % _NBLK
            j_ = s_ * 2 + g
            blk = x_refs[r][pl.ds(j_, 1), pl.ds(b_, 1), :]
            l_ = jnp.min(jnp.where(blk == m, lio, _BIG))
            blk2 = jnp.where(lio == l_, _NEG, blk)
            x_refs[r][pl.ds(j_, 1), pl.ds(b_, 1), :] = blk2
            newm = jnp.max(blk2)
            Mgs[r] = jnp.where((jio == j_) & (bio == b_), newm, Mgs[r])
            picks[r].append((m, b_ * _BLK + l_, j_))

    picks0 = [[] for _ in range(_R)]
    Mg0 = [jnp.where(jio % 2 == 0, Ms[r], _NEG) for r in range(_R)]
    for _ in range(8):
        pick_round(Mg0, 0, picks0)

    for t in range(8):
        for r in range(_R):
            v_ = picks0[r][t][1]
            b_ = v_ // _BLK
            l_ = v_ % _BLK
            sl = x_refs[r][:, pl.ds(b_, 1), :]
            sl2 = sl - jnp.where(odd3 & (lio8 == l_), half, jnp.float32(0.0))
            x_refs[r][:, pl.ds(b_, 1), :] = sl2
            ncol = jnp.max(sl2, axis=2)
            Ms[r] = jnp.where((jio % 2 == 1) & (bio == b_), ncol, Ms[r])

    picks1 = [[] for _ in range(_R)]
    Mg1 = [jnp.where(jio % 2 == 1, Ms[r], _NEG) for r in range(_R)]
    for _ in range(8):
        pick_round(Mg1, 1, picks1)

    i16 = lax.broadcasted_iota(jnp.int32, (1, 16), 1)
    for r in range(_R):
        vv = jnp.zeros((1, 16), jnp.float32)
        iv = jnp.zeros((1, 16), jnp.int32)
        bv = jnp.zeros((1, 16), jnp.int32)
        for k in range(8):
            for g, pk in ((0, picks0[r][k]), (1, picks1[r][k])):
                slot = 2 * k + g
                vv = jnp.where(i16 == slot, pk[0], vv)
                iv = jnp.where(i16 == slot, pk[1], iv)
                bv = jnp.where(i16 == slot, pk[2], bv)
        vals_ref[r] = vv
        idx_ref[r] = iv
        beams_ref[r] = bv


@jax.jit
def _run(lp4, sc3):
    bsz = lp4.shape[0]
    out = pl.pallas_call(
        _rows_kernel,
        grid=(bsz // _R,),
        in_specs=[
            pl.BlockSpec((_R, 8, _NBLK, _BLK), lambda i: (i, 0, 0, 0)),
            pl.BlockSpec((_R, 1, 8), lambda i: (i, 0, 0)),
        ],
        out_specs=[
            pl.BlockSpec((_R, 1, 16), lambda i: (i, 0, 0)),
            pl.BlockSpec((_R, 1, 16), lambda i: (i, 0, 0)),
            pl.BlockSpec((_R, 1, 16), lambda i: (i, 0, 0)),
        ],
        out_shape=[
            jax.ShapeDtypeStruct((bsz, 1, 16), jnp.float32),
            jax.ShapeDtypeStruct((bsz, 1, 16), jnp.int32),
            jax.ShapeDtypeStruct((bsz, 1, 16), jnp.int32),
        ],
        scratch_shapes=[pltpu.VMEM((8, _NBLK, _BLK), jnp.float32)] * _R,
    )(lp4, sc3)
    return out


def kernel(step, lprobs, scores):
    bsz, beam_size, vocab = lprobs.shape
    lp4 = lprobs.reshape(bsz, beam_size, _NBLK, _BLK)
    sc = lax.dynamic_slice_in_dim(scores, step - 1, 1, axis=2)
    sc3 = sc.reshape(bsz, 1, beam_size)
    vv, iv, bv = _run(lp4, sc3)
    return (vv.reshape(bsz, 16), iv.reshape(bsz, 16), bv.reshape(bsz, 16))
